# baseline (speedup 1.0000x reference)
import functools

import jax
import jax.numpy as jnp
from jax import lax
from jax.experimental import pallas as pl
from jax.experimental.pallas import tpu as pltpu
from jax.experimental.pallas import tpu_sc as plsc

_BS, _SEQ, _H, _D = 8, 2048, 16, 128
_ROWS = _BS * _SEQ
_NC = 2
_NW = 32
_RPW = _ROWS // _NW
_TPB = _SEQ // _RPW
_CHUNKS = (512, 256, 128, 64, 32, 16, 8, 4, 2, 1)


def _copy_ranged(src_k, src_v, dst_k, dst_v, s0, d0, n):
    pos = jnp.int32(0)
    rem = n
    for c in _CHUNKS:
        take = rem >= c

        @pl.when(take)
        def _():
            pltpu.sync_copy(src_k.at[pl.ds(s0 + pos, c)],
                            dst_k.at[pl.ds(d0 + pos, c)])
            pltpu.sync_copy(src_v.at[pl.ds(s0 + pos, c)],
                            dst_v.at[pl.ds(d0 + pos, c)])

        step = jnp.where(take, jnp.int32(c), jnp.int32(0))
        pos = pos + step
        rem = rem - step


@functools.partial(
    pl.kernel,
    out_type=(
        jax.ShapeDtypeStruct((_ROWS, _H, _D), jnp.float32),
        jax.ShapeDtypeStruct((_ROWS, _H, _D), jnp.float32),
        jax.ShapeDtypeStruct((16,), jnp.int32),
    ),
    mesh=plsc.VectorSubcoreMesh(core_axis_name="c", subcore_axis_name="s"),
    scratch_types=[pltpu.VMEM((16,), jnp.int32)],
    compiler_params=pltpu.CompilerParams(needs_layout_passes=False),
)
def _fill(ks, vs, sl, kc, vc, ko, vo, co, slv):
    w = lax.axis_index("s") * _NC + lax.axis_index("c")

    slv[...] = jnp.zeros((16,), jnp.int32)
    pltpu.sync_copy(sl, slv.at[pl.ds(0, _BS)])
    x = slv[...]
    lanes = lax.iota(jnp.int32, 16)
    xm = jnp.where(lanes < _BS, x, 0)

    i = w // _TPB
    t0 = (w % _TPB) * _RPW
    seg_len = jnp.sum(jnp.where(lanes == i, xm, 0))
    seg_off = jnp.sum(jnp.where(lanes < i, xm, 0))
    total = jnp.sum(xm)

    n = jnp.clip(seg_len - t0, 0, _RPW)
    _copy_ranged(ks, vs, ko, vo, i * _SEQ + t0, seg_off + t0, n)

    ts = jnp.maximum(total, w * _RPW)
    n_tail = jnp.clip((w + 1) * _RPW - ts, 0, _RPW)
    _copy_ranged(kc, vc, ko, vo, ts, ts, n_tail)

    @pl.when(w == 0)
    def _():
        z = jnp.cumsum(xm) - xm
        slv[...] = z
        pltpu.sync_copy(slv, co)


def kernel(key_states, value_states, seq_lens, k_cache, v_cache):
    ks = key_states.reshape(_ROWS, _H, _D)
    vs = value_states.reshape(_ROWS, _H, _D)
    ko, vo, co = _fill(ks, vs, seq_lens, k_cache, v_cache)
    return ko, vo, seq_lens.astype(jnp.int32), co[: _BS + 1]


# speedup vs baseline: 22.4250x; 22.4250x over previous
import functools

import jax
import jax.numpy as jnp
from jax import lax
from jax.experimental import pallas as pl
from jax.experimental.pallas import tpu as pltpu
from jax.experimental.pallas import tpu_sc as plsc

_BS, _SEQ, _H, _D = 8, 2048, 16, 128
_ROWS = _BS * _SEQ
_NC = 2
_NW = 32
_RPW = _ROWS // _NW
_TPB = _SEQ // _RPW
_CH = 24
_REM_CHUNKS = (16, 8, 4, 2, 1)


def _staged_copy(src_k, src_v, dst_k, dst_v, s0, d0, n, kb, vb, sem):
    nf = n // _CH

    def body(j, carry):
        s = s0 + j * _CH
        d = d0 + j * _CH
        g1 = pltpu.async_copy(src_k.at[pl.ds(s, _CH)], kb, sem)
        g2 = pltpu.async_copy(src_v.at[pl.ds(s, _CH)], vb, sem)
        g1.wait()
        g2.wait()
        s1 = pltpu.async_copy(kb, dst_k.at[pl.ds(d, _CH)], sem)
        s2 = pltpu.async_copy(vb, dst_v.at[pl.ds(d, _CH)], sem)
        s1.wait()
        s2.wait()
        return carry

    lax.fori_loop(0, nf, body, 0)

    base = s0 + nf * _CH
    dbase = d0 + nf * _CH
    rem = n - nf * _CH
    pos = jnp.int32(0)
    for c in _REM_CHUNKS:
        take = rem >= c

        @pl.when(take)
        def _():
            s = base + pos
            d = dbase + pos
            g1 = pltpu.async_copy(src_k.at[pl.ds(s, c)], kb.at[pl.ds(0, c)], sem)
            g2 = pltpu.async_copy(src_v.at[pl.ds(s, c)], vb.at[pl.ds(0, c)], sem)
            g1.wait()
            g2.wait()
            s1 = pltpu.async_copy(kb.at[pl.ds(0, c)], dst_k.at[pl.ds(d, c)], sem)
            s2 = pltpu.async_copy(vb.at[pl.ds(0, c)], dst_v.at[pl.ds(d, c)], sem)
            s1.wait()
            s2.wait()

        step = jnp.where(take, jnp.int32(c), jnp.int32(0))
        pos = pos + step
        rem = rem - step


@functools.partial(
    pl.kernel,
    out_type=(
        jax.ShapeDtypeStruct((_ROWS, _H, _D), jnp.float32),
        jax.ShapeDtypeStruct((_ROWS, _H, _D), jnp.float32),
        jax.ShapeDtypeStruct((16,), jnp.int32),
    ),
    mesh=plsc.VectorSubcoreMesh(core_axis_name="c", subcore_axis_name="s"),
    scratch_types=[
        pltpu.VMEM((16,), jnp.int32),
        pltpu.VMEM((_CH, _H, _D), jnp.float32),
        pltpu.VMEM((_CH, _H, _D), jnp.float32),
        pltpu.SemaphoreType.DMA,
    ],
    compiler_params=pltpu.CompilerParams(needs_layout_passes=False),
)
def _fill(ks, vs, sl, kc, vc, ko, vo, co, slv, kb, vb, sem):
    w = lax.axis_index("s") * _NC + lax.axis_index("c")

    slv[...] = jnp.zeros((16,), jnp.int32)
    pltpu.sync_copy(sl, slv.at[pl.ds(0, _BS)])
    x = slv[...]
    lanes = lax.iota(jnp.int32, 16)
    xm = jnp.where(lanes < _BS, x, 0)

    i = w // _TPB
    t0 = (w % _TPB) * _RPW
    seg_len = jnp.sum(jnp.where(lanes == i, xm, 0))
    seg_off = jnp.sum(jnp.where(lanes < i, xm, 0))
    total = jnp.sum(xm)

    n = jnp.clip(seg_len - t0, 0, _RPW)
    _staged_copy(ks, vs, ko, vo, i * _SEQ + t0, seg_off + t0, n, kb, vb, sem)

    ts = jnp.maximum(total, w * _RPW)
    n_tail = jnp.clip((w + 1) * _RPW - ts, 0, _RPW)
    _staged_copy(kc, vc, ko, vo, ts, ts, n_tail, kb, vb, sem)

    @pl.when(w == 0)
    def _():
        z = jnp.cumsum(xm) - xm
        slv[...] = z
        pltpu.sync_copy(slv, co)


def kernel(key_states, value_states, seq_lens, k_cache, v_cache):
    ks = key_states.reshape(_ROWS, _H, _D)
    vs = value_states.reshape(_ROWS, _H, _D)
    ko, vo, co = _fill(ks, vs, seq_lens, k_cache, v_cache)
    return ko, vo, seq_lens.astype(jnp.int32), co[: _BS + 1]


# speedup vs baseline: 37.6942x; 1.6809x over previous
import functools

import jax
import jax.numpy as jnp
from jax import lax
from jax.experimental import pallas as pl
from jax.experimental.pallas import tpu as pltpu
from jax.experimental.pallas import tpu_sc as plsc

_BS, _SEQ, _H, _D = 8, 2048, 16, 128
_ROWS = _BS * _SEQ
_NC = 2
_NW = 32
_RPW = _ROWS // _NW
_G = 16
_REM = (8, 4, 2, 1)


@functools.partial(
    pl.kernel,
    out_type=(
        jax.ShapeDtypeStruct((_ROWS, _H, _D), jnp.float32),
        jax.ShapeDtypeStruct((_ROWS, _H, _D), jnp.float32),
        jax.ShapeDtypeStruct((16,), jnp.int32),
    ),
    mesh=plsc.VectorSubcoreMesh(core_axis_name="c", subcore_axis_name="s"),
    scratch_types=[
        pltpu.VMEM((16,), jnp.int32),
        pltpu.VMEM((16,), jnp.int32),
        pltpu.VMEM((_G, _H, _D), jnp.float32),
        pltpu.VMEM((_G, _H, _D), jnp.float32),
        pltpu.VMEM((_G, _H, _D), jnp.float32),
        pltpu.SemaphoreType.DMA,
    ],
    compiler_params=pltpu.CompilerParams(needs_layout_passes=False),
)
def _fill(ks, vs, sl, kc, vc, ko, vo, co, slv, zxv, kb, vb, zb, sem):
    w = lax.axis_index("s") * _NC + lax.axis_index("c")
    o0 = w * _RPW

    slv[...] = jnp.zeros((16,), jnp.int32)
    pltpu.sync_copy(sl, slv.at[pl.ds(0, _BS)])
    x = slv[...]
    lanes = lax.iota(jnp.int32, 16)
    xm = jnp.where(lanes < _BS, x, 0)
    zx = jnp.cumsum(xm) - xm
    zxv[...] = zx
    total = jnp.sum(xm)
    cs = [jnp.sum(jnp.where(lanes == j, zx, 0)) for j in range(1, _BS + 1)]

    valid_end = jnp.clip(total - o0, 0, _RPW)
    nfull = valid_end // _G
    rem = valid_end - nfull * _G

    def src_for(r0):
        rvec = r0 + lanes
        b = jnp.zeros((16,), jnp.int32)
        for c in cs:
            b = b + jnp.where(rvec >= c, 1, 0)
        bb = jnp.where(b < _BS, b, 0)
        base = plsc.load_gather(zxv, [bb])
        src = bb * _SEQ + (rvec - base)
        return jnp.where(b < _BS, src, 0)

    def gbody(g, carry):
        r0 = o0 + g * _G
        src = src_for(r0)
        gk = pltpu.async_copy(ks.at[src], kb, sem)
        gv = pltpu.async_copy(vs.at[src], vb, sem)
        gk.wait()
        gv.wait()
        sk = pltpu.async_copy(kb, ko.at[pl.ds(r0, _G)], sem)
        sv = pltpu.async_copy(vb, vo.at[pl.ds(r0, _G)], sem)
        sk.wait()
        sv.wait()
        return carry

    lax.fori_loop(0, nfull, gbody, 0)

    bnd = o0 + nfull * _G

    @pl.when(rem > 0)
    def _():
        src = src_for(bnd)
        gk = pltpu.async_copy(ks.at[src], kb, sem)
        gv = pltpu.async_copy(vs.at[src], vb, sem)
        gk.wait()
        gv.wait()

    pos = jnp.int32(0)
    rr = rem
    for c in _REM:
        take = rr >= c

        @pl.when(take)
        def _():
            sk = pltpu.async_copy(kb.at[pl.ds(pos, c)], ko.at[pl.ds(bnd + pos, c)], sem)
            sv = pltpu.async_copy(vb.at[pl.ds(pos, c)], vo.at[pl.ds(bnd + pos, c)], sem)
            sk.wait()
            sv.wait()

        st = jnp.where(take, jnp.int32(c), jnp.int32(0))
        pos = pos + st
        rr = rr - st

    gz = pltpu.async_copy(kc.at[pl.ds(0, _G)], zb, sem)
    gz.wait()
    tstart = o0 + valid_end
    nt = _RPW - valid_end
    ntf = nt // _G

    def tbody(t, carry):
        d = tstart + t * _G
        sk = pltpu.async_copy(zb, ko.at[pl.ds(d, _G)], sem)
        sv = pltpu.async_copy(zb, vo.at[pl.ds(d, _G)], sem)
        sk.wait()
        sv.wait()
        return carry

    lax.fori_loop(0, ntf, tbody, 0)

    tpos = tstart + ntf * _G
    rt = nt - ntf * _G
    for c in _REM:
        take = rt >= c

        @pl.when(take)
        def _():
            sk = pltpu.async_copy(zb.at[pl.ds(0, c)], ko.at[pl.ds(tpos, c)], sem)
            sv = pltpu.async_copy(zb.at[pl.ds(0, c)], vo.at[pl.ds(tpos, c)], sem)
            sk.wait()
            sv.wait()

        st = jnp.where(take, jnp.int32(c), jnp.int32(0))
        tpos = tpos + st
        rt = rt - st

    @pl.when(w == 0)
    def _():
        slv[...] = zx
        pltpu.sync_copy(slv, co)


def kernel(key_states, value_states, seq_lens, k_cache, v_cache):
    ks = key_states.reshape(_ROWS, _H, _D)
    vs = value_states.reshape(_ROWS, _H, _D)
    ko, vo, co = _fill(ks, vs, seq_lens, k_cache, v_cache)
    return ko, vo, seq_lens.astype(jnp.int32), co[: _BS + 1]
